# Initial kernel scaffold; baseline (speedup 1.0000x reference)
import jax
import jax.numpy as jnp
from jax.experimental import pallas as pl


def kernel(x, edge_index, edge_weight, W1, b1, W2, b2):
    raise NotImplementedError("write your pallas kernel here")


# speedup vs baseline: 15.7068x; 15.7068x over previous
import functools

import jax
import jax.numpy as jnp
from jax import lax
from jax.experimental import pallas as pl
from jax.experimental.pallas import tpu as pltpu
from jax.experimental.pallas import tpu_sc as plsc

N = 10000
E = 320000
NFEAT = 128
NHID = 128
NCLASS = 64

NC = 2
NS = 16
L = 16
NW = NC * NS

K = 128
NCHW = 80
EW = NCHW * K
EPAD = NW * EW

NPAD = 10240
RPS = NPAD // NS
NZ = RPS // K

F = 128


def _mesh():
  return plsc.VectorSubcoreMesh(
      core_axis_name="c", subcore_axis_name="s", num_cores=NC, num_subcores=NS
  )


def _sc_params():
  return pltpu.CompilerParams(needs_layout_passes=False)


def _deg_body(colh, wh, out0, out1, colv, wv, zb, acc):
  c = lax.axis_index("c")
  s = lax.axis_index("s")
  wid = s * NC + c
  base = wid * NCHW

  pltpu.sync_copy(colh.at[pl.ds(base, NCHW)], colv)
  pltpu.sync_copy(wh.at[pl.ds(base, NCHW)], wv)

  def zbody(i, _):
    zb[pl.ds(i * L, L)] = jnp.zeros((L,), jnp.float32)
    return 0

  lax.fori_loop(0, K // L, zbody, 0)
  for t in range(NZ):
    pltpu.sync_copy(zb, acc.at[pl.ds(s * RPS + t * K, K)])
  plsc.subcore_barrier()

  def chunk(j, _):
    pltpu.sync_copy(wv.at[j], acc.at[colv.at[j]], add=True)
    return 0

  lax.fori_loop(0, NCHW, chunk, 0)
  plsc.subcore_barrier()

  sl = pl.ds(s * RPS, RPS)

  @pl.when(c == 0)
  def _():
    pltpu.sync_copy(acc.at[sl], out0.at[sl])

  @pl.when(c == 1)
  def _():
    pltpu.sync_copy(acc.at[sl], out1.at[sl])


@jax.jit
def _deg(colh, wh):
  fn = pl.kernel(
      _deg_body,
      out_type=(
          jax.ShapeDtypeStruct((NPAD,), jnp.float32),
          jax.ShapeDtypeStruct((NPAD,), jnp.float32),
      ),
      mesh=_mesh(),
      scratch_types=[
          pltpu.VMEM((NCHW, K), jnp.int32),
          pltpu.VMEM((NCHW, K), jnp.float32),
          pltpu.VMEM((K,), jnp.float32),
          pltpu.VMEM_SHARED((NPAD,), jnp.float32),
      ],
      compiler_params=_sc_params(),
  )
  return fn(colh, wh)


def _agg_body(hp, rowh, colh, wh1, out, rowv, colv, wv1, gbuf, acc, sem):
  c = lax.axis_index("c")
  s = lax.axis_index("s")
  wid = s * NC + c
  base = wid * NCHW

  pltpu.sync_copy(rowh.at[pl.ds(base, NCHW)], rowv)
  pltpu.sync_copy(colh.at[pl.ds(base, NCHW)], colv)
  pltpu.sync_copy(wh1.at[pl.ds(wid * EW, EW)], wv1)

  def zbody(i, _):
    for f in range(F // L):
      gbuf[i, pl.ds(f * L, L)] = jnp.zeros((L,), jnp.float32)
    return 0

  lax.fori_loop(0, K, zbody, 0)
  for t in range(NZ):
    pltpu.sync_copy(gbuf, acc.at[pl.ds(s * RPS + t * K, K)])
  plsc.subcore_barrier()

  def chunk(j, _):
    pltpu.async_copy(hp.at[rowv.at[j]], gbuf, sem).wait()

    def edge(e, _):
      w16 = plsc.load_gather(wv1, [jnp.full((L,), j * K + e, jnp.int32)])
      for f in range(F // L):
        sl = pl.ds(f * L, L)
        gbuf[e, sl] = gbuf[e, sl] * w16
      return 0

    lax.fori_loop(0, K, edge, 0)
    pltpu.sync_copy(gbuf, acc.at[colv.at[j]], add=True)
    return 0

  lax.fori_loop(0, NCHW, chunk, 0)
  plsc.subcore_barrier()

  sl = pl.ds(s * RPS, RPS)
  pltpu.sync_copy(acc.at[sl], out.at[c, sl])


@jax.jit
def _agg(hp, rowh, colh, wh1):
  fn = pl.kernel(
      _agg_body,
      out_type=jax.ShapeDtypeStruct((NC, NPAD, F), jnp.float32),
      mesh=_mesh(),
      scratch_types=[
          pltpu.VMEM((NCHW, K), jnp.int32),
          pltpu.VMEM((NCHW, K), jnp.int32),
          pltpu.VMEM((EW,), jnp.float32),
          pltpu.VMEM((K, F), jnp.float32),
          pltpu.VMEM_SHARED((NPAD, F), jnp.float32),
          pltpu.SemaphoreType.DMA,
      ],
      compiler_params=_sc_params(),
  )
  return fn(hp, rowh, colh, wh1)


def _dis(dp_ref):
  return lax.rsqrt(1.0 + dp_ref[:, 0:1] + dp_ref[:, 1:2])


def _tca_body(dp, x, w1, o):
  o[...] = _dis(dp) * jnp.dot(
      x[...], w1[...], preferred_element_type=jnp.float32
  )


def _tcb_body(dp, agg, hp, b1, w2, o):
  dis = _dis(dp)
  z = dis * (agg[0] + agg[1] + hp[...]) + b1[...][None, :]
  a = jnp.maximum(z, 0.0)
  h2 = dis * jnp.dot(a, w2[...], preferred_element_type=jnp.float32)
  o[...] = jnp.concatenate([h2, jnp.zeros_like(h2)], axis=1)


def _tcc_body(dp, agg, hp, b2, o):
  dis = _dis(dp)
  o[...] = (
      dis * (agg[0, :, 0:NCLASS] + agg[1, :, 0:NCLASS] + hp[:, 0:NCLASS])
      + b2[...][None, :]
  )


def _agg_spec():
  return pl.BlockSpec((NC, N, F), lambda i: (0, 0, 0))


@jax.jit
def _tca(dp, x, w1):
  return pl.pallas_call(
      _tca_body,
      out_shape=jax.ShapeDtypeStruct((N, NHID), jnp.float32),
  )(dp, x, w1)


@jax.jit
def _tcb(dp, agg, hp, b1, w2):
  return pl.pallas_call(
      _tcb_body,
      grid=(1,),
      in_specs=[
          pl.BlockSpec((N, 2), lambda i: (0, 0)),
          _agg_spec(),
          pl.BlockSpec((N, NHID), lambda i: (0, 0)),
          pl.BlockSpec((NHID,), lambda i: (0,)),
          pl.BlockSpec((NHID, NCLASS), lambda i: (0, 0)),
      ],
      out_specs=pl.BlockSpec((N, F), lambda i: (0, 0)),
      out_shape=jax.ShapeDtypeStruct((N, F), jnp.float32),
  )(dp, agg, hp, b1, w2)


@jax.jit
def _tcc(dp, agg, hp, b2):
  return pl.pallas_call(
      _tcc_body,
      grid=(1,),
      in_specs=[
          pl.BlockSpec((N, 2), lambda i: (0, 0)),
          _agg_spec(),
          pl.BlockSpec((N, F), lambda i: (0, 0)),
          pl.BlockSpec((NCLASS,), lambda i: (0,)),
      ],
      out_specs=pl.BlockSpec((N, NCLASS), lambda i: (0, 0)),
      out_shape=jax.ShapeDtypeStruct((N, NCLASS), jnp.float32),
  )(dp, agg, hp, b2)


def kernel(x, edge_index, edge_weight, W1, b1, W2, b2):
  row = edge_index[0].astype(jnp.int32)
  col = edge_index[1].astype(jnp.int32)
  w = edge_weight.astype(jnp.float32)

  pad = EPAD - E
  pidx = (jnp.arange(pad, dtype=jnp.int32) * 997) % N
  row_p = jnp.concatenate([row, pidx]).reshape(NW * NCHW, K)
  col_p = jnp.concatenate([col, pidx]).reshape(NW * NCHW, K)
  w_flat = jnp.concatenate([w, jnp.zeros((pad,), jnp.float32)])
  w_p = w_flat.reshape(NW * NCHW, K)

  d0, d1 = _deg(col_p, w_p)
  dp = jnp.stack([d0[:N], d1[:N]], axis=1)

  h1p = _tca(dp, x, W1)
  agg1 = _agg(h1p, row_p, col_p, w_flat)
  h2p = _tcb(dp, agg1, h1p, b1, W2)
  agg2 = _agg(h2p, row_p, col_p, w_flat)
  out = _tcc(dp, agg2, h2p, b2)
  return out


# speedup vs baseline: 19.2204x; 1.2237x over previous
import functools

import jax
import jax.numpy as jnp
from jax import lax
from jax.experimental import pallas as pl
from jax.experimental.pallas import tpu as pltpu
from jax.experimental.pallas import tpu_sc as plsc

N = 10000
E = 320000
NFEAT = 128
NHID = 128
NCLASS = 64

NC = 2
NS = 16
L = 16
NW = NC * NS

K = 128
NCHW = 80
EW = NCHW * K
EPAD = NW * EW

NPAD = 10240
RPS = NPAD // NS
NZ = RPS // K

F = 128


def _mesh():
  return plsc.VectorSubcoreMesh(
      core_axis_name="c", subcore_axis_name="s", num_cores=NC, num_subcores=NS
  )


def _sc_params():
  return pltpu.CompilerParams(needs_layout_passes=False)


def _deg_body(colh, wh, out0, out1, colv, wv, zb, acc):
  c = lax.axis_index("c")
  s = lax.axis_index("s")
  wid = s * NC + c
  base = wid * NCHW

  pltpu.sync_copy(colh.at[pl.ds(base, NCHW)], colv)
  pltpu.sync_copy(wh.at[pl.ds(base, NCHW)], wv)

  def zbody(i, _):
    zb[pl.ds(i * L, L)] = jnp.zeros((L,), jnp.float32)
    return 0

  lax.fori_loop(0, K // L, zbody, 0)
  for t in range(NZ):
    pltpu.sync_copy(zb, acc.at[pl.ds(s * RPS + t * K, K)])
  plsc.subcore_barrier()

  def chunk(j, _):
    pltpu.sync_copy(wv.at[j], acc.at[colv.at[j]], add=True)
    return 0

  lax.fori_loop(0, NCHW, chunk, 0)
  plsc.subcore_barrier()

  sl = pl.ds(s * RPS, RPS)

  @pl.when(c == 0)
  def _():
    pltpu.sync_copy(acc.at[sl], out0.at[sl])

  @pl.when(c == 1)
  def _():
    pltpu.sync_copy(acc.at[sl], out1.at[sl])


@jax.jit
def _deg(colh, wh):
  fn = pl.kernel(
      _deg_body,
      out_type=(
          jax.ShapeDtypeStruct((NPAD,), jnp.float32),
          jax.ShapeDtypeStruct((NPAD,), jnp.float32),
      ),
      mesh=_mesh(),
      scratch_types=[
          pltpu.VMEM((NCHW, K), jnp.int32),
          pltpu.VMEM((NCHW, K), jnp.float32),
          pltpu.VMEM((K,), jnp.float32),
          pltpu.VMEM_SHARED((NPAD,), jnp.float32),
      ],
      compiler_params=_sc_params(),
  )
  return fn(colh, wh)


NBUF = 2


def _agg_body(
    hp, rowh, colh, wh, out, rowv, colb, wb0, wb1, gb, acc, gsem, ssem, isem
):
  c = lax.axis_index("c")
  s = lax.axis_index("s")
  wid = s * NC + c
  base = wid * NCHW
  wbs = (wb0, wb1)

  pltpu.sync_copy(rowh.at[pl.ds(base, NCHW)], rowv)

  def zbody(i, _):
    for f in range(F // L):
      gb[0, i, pl.ds(f * L, L)] = jnp.zeros((L,), jnp.float32)
    return 0

  lax.fori_loop(0, K, zbody, 0)
  for t in range(NZ):
    pltpu.sync_copy(gb.at[0], acc.at[pl.ds(s * RPS + t * K, K)])
  plsc.subcore_barrier()

  def g_start(b, cj):
    pltpu.async_copy(hp.at[rowv.at[cj]], gb.at[b], gsem.at[b])

  def g_wait(b, cj):
    pltpu.make_async_copy(hp.at[rowv.at[cj]], gb.at[b], gsem.at[b]).wait()

  def s_start(b, cj):
    pltpu.async_copy(gb.at[b], acc.at[colb.at[b]], ssem.at[b], add=True)

  def s_wait(b, cj):
    pltpu.make_async_copy(gb.at[b], acc.at[colb.at[b]], ssem.at[b]).wait()

  def i_start(b, cj):
    pltpu.async_copy(colh.at[base + cj], colb.at[b], isem.at[b])
    pltpu.async_copy(wh.at[base + cj], wbs[b], isem.at[b])

  def i_wait(b, cj):
    pltpu.make_async_copy(colh.at[base + cj], colb.at[b], isem.at[b]).wait()
    pltpu.make_async_copy(wh.at[base + cj], wbs[b], isem.at[b]).wait()

  i_start(0, 0)
  g_start(0, 0)

  def pair(t, _):
    j = t * NBUF
    for b in range(NBUF):
      cj = j + b
      bp = (b + NBUF - 1) % NBUF

      g_wait(b, cj)
      i_wait(b, cj)

      def edge(e, _):
        w16 = plsc.load_gather(wbs[b], [jnp.full((L,), e, jnp.int32)])
        for f in range(F // L):
          sl = pl.ds(f * L, L)
          gb[b, e, sl] = gb[b, e, sl] * w16
        return 0

      lax.fori_loop(0, K, edge, 0, unroll=2)

      s_start(b, cj)

      @pl.when(cj >= 1)
      def _():
        s_wait(bp, cj - 1)

      @pl.when(cj + 1 < NCHW)
      def _():
        i_start(bp, cj + 1)
        g_start(bp, cj + 1)

    return 0

  lax.fori_loop(0, NCHW // NBUF, pair, 0)
  s_wait((NCHW - 1) % NBUF, NCHW - 1)
  plsc.subcore_barrier()

  sl = pl.ds(s * RPS, RPS)
  pltpu.sync_copy(acc.at[sl], out.at[c, sl])


@jax.jit
def _agg(hp, rowh, colh, wh):
  fn = pl.kernel(
      _agg_body,
      out_type=jax.ShapeDtypeStruct((NC, NPAD, F), jnp.float32),
      mesh=_mesh(),
      scratch_types=[
          pltpu.VMEM((NCHW, K), jnp.int32),
          pltpu.VMEM((NBUF, K), jnp.int32),
          pltpu.VMEM((K,), jnp.float32),
          pltpu.VMEM((K,), jnp.float32),
          pltpu.VMEM((NBUF, K, F), jnp.float32),
          pltpu.VMEM_SHARED((NPAD, F), jnp.float32),
          pltpu.SemaphoreType.DMA((NBUF,)),
          pltpu.SemaphoreType.DMA((NBUF,)),
          pltpu.SemaphoreType.DMA((NBUF,)),
      ],
      compiler_params=_sc_params(),
  )
  return fn(hp, rowh, colh, wh)


def _dis(dp_ref):
  return lax.rsqrt(1.0 + dp_ref[:, 0:1] + dp_ref[:, 1:2])


def _tca_body(dp, x, w1, o):
  o[...] = _dis(dp) * jnp.dot(
      x[...], w1[...], preferred_element_type=jnp.float32
  )


def _tcb_body(dp, agg, hp, b1, w2, o):
  dis = _dis(dp)
  z = dis * (agg[0] + agg[1] + hp[...]) + b1[...][None, :]
  a = jnp.maximum(z, 0.0)
  h2 = dis * jnp.dot(a, w2[...], preferred_element_type=jnp.float32)
  o[...] = jnp.concatenate([h2, jnp.zeros_like(h2)], axis=1)


def _tcc_body(dp, agg, hp, b2, o):
  dis = _dis(dp)
  o[...] = (
      dis * (agg[0, :, 0:NCLASS] + agg[1, :, 0:NCLASS] + hp[:, 0:NCLASS])
      + b2[...][None, :]
  )


def _agg_spec():
  return pl.BlockSpec((NC, N, F), lambda i: (0, 0, 0))


@jax.jit
def _tca(dp, x, w1):
  return pl.pallas_call(
      _tca_body,
      out_shape=jax.ShapeDtypeStruct((N, NHID), jnp.float32),
  )(dp, x, w1)


@jax.jit
def _tcb(dp, agg, hp, b1, w2):
  return pl.pallas_call(
      _tcb_body,
      grid=(1,),
      in_specs=[
          pl.BlockSpec((N, 2), lambda i: (0, 0)),
          _agg_spec(),
          pl.BlockSpec((N, NHID), lambda i: (0, 0)),
          pl.BlockSpec((NHID,), lambda i: (0,)),
          pl.BlockSpec((NHID, NCLASS), lambda i: (0, 0)),
      ],
      out_specs=pl.BlockSpec((N, F), lambda i: (0, 0)),
      out_shape=jax.ShapeDtypeStruct((N, F), jnp.float32),
  )(dp, agg, hp, b1, w2)


@jax.jit
def _tcc(dp, agg, hp, b2):
  return pl.pallas_call(
      _tcc_body,
      grid=(1,),
      in_specs=[
          pl.BlockSpec((N, 2), lambda i: (0, 0)),
          _agg_spec(),
          pl.BlockSpec((N, F), lambda i: (0, 0)),
          pl.BlockSpec((NCLASS,), lambda i: (0,)),
      ],
      out_specs=pl.BlockSpec((N, NCLASS), lambda i: (0, 0)),
      out_shape=jax.ShapeDtypeStruct((N, NCLASS), jnp.float32),
  )(dp, agg, hp, b2)


def kernel(x, edge_index, edge_weight, W1, b1, W2, b2):
  row = edge_index[0].astype(jnp.int32)
  col = edge_index[1].astype(jnp.int32)
  w = edge_weight.astype(jnp.float32)

  pad = EPAD - E
  pidx = (jnp.arange(pad, dtype=jnp.int32) * 997) % N
  row_p = jnp.concatenate([row, pidx]).reshape(NW * NCHW, K)
  col_p = jnp.concatenate([col, pidx]).reshape(NW * NCHW, K)
  w_p = jnp.concatenate([w, jnp.zeros((pad,), jnp.float32)]).reshape(
      NW * NCHW, K
  )

  d0, d1 = _deg(col_p, w_p)
  dp = jnp.stack([d0[:N], d1[:N]], axis=1)

  h1p = _tca(dp, x, W1)
  agg1 = _agg(h1p, row_p, col_p, w_p)
  h2p = _tcb(dp, agg1, h1p, b1, W2)
  agg2 = _agg(h2p, row_p, col_p, w_p)
  out = _tcc(dp, agg2, h2p, b2)
  return out


# speedup vs baseline: 21.0934x; 1.0974x over previous
import functools

import jax
import jax.numpy as jnp
from jax import lax
from jax.experimental import pallas as pl
from jax.experimental.pallas import tpu as pltpu
from jax.experimental.pallas import tpu_sc as plsc

N = 10000
E = 320000
NFEAT = 128
NHID = 128
NCLASS = 64

NC = 2
NS = 16
L = 16
NW = NC * NS

K = 128
NCHW = 80
EW = NCHW * K
EPAD = NW * EW

NPAD = 10240
RPS = NPAD // NS
NZ = RPS // K

F = 128


def _mesh():
  return plsc.VectorSubcoreMesh(
      core_axis_name="c", subcore_axis_name="s", num_cores=NC, num_subcores=NS
  )


def _sc_params(tc_tiling=True):
  return pltpu.CompilerParams(
      needs_layout_passes=False, use_tc_tiling_on_sc=tc_tiling
  )


def _deg_body(colh, wh, out0, out1, colv, wv, zb, acc):
  c = lax.axis_index("c")
  s = lax.axis_index("s")
  wid = s * NC + c
  base = wid * NCHW

  pltpu.sync_copy(colh.at[pl.ds(base, NCHW)], colv)
  pltpu.sync_copy(wh.at[pl.ds(base, NCHW)], wv)

  def zbody(i, _):
    zb[pl.ds(i * L, L)] = jnp.zeros((L,), jnp.float32)
    return 0

  lax.fori_loop(0, K // L, zbody, 0)
  for t in range(NZ):
    pltpu.sync_copy(zb, acc.at[pl.ds(s * RPS + t * K, K)])
  plsc.subcore_barrier()

  def chunk(j, _):
    pltpu.sync_copy(wv.at[j], acc.at[colv.at[j]], add=True)
    return 0

  lax.fori_loop(0, NCHW, chunk, 0)
  plsc.subcore_barrier()

  sl = pl.ds(s * RPS, RPS)

  @pl.when(c == 0)
  def _():
    pltpu.sync_copy(acc.at[sl], out0.at[sl])

  @pl.when(c == 1)
  def _():
    pltpu.sync_copy(acc.at[sl], out1.at[sl])


@jax.jit
def _deg(colh, wh):
  fn = pl.kernel(
      _deg_body,
      out_type=(
          jax.ShapeDtypeStruct((NPAD,), jnp.float32),
          jax.ShapeDtypeStruct((NPAD,), jnp.float32),
      ),
      mesh=_mesh(),
      scratch_types=[
          pltpu.VMEM((NCHW, K), jnp.int32),
          pltpu.VMEM((NCHW, K), jnp.float32),
          pltpu.VMEM((K,), jnp.float32),
          pltpu.VMEM_SHARED((NPAD,), jnp.float32),
      ],
      compiler_params=_sc_params(),
  )
  return fn(colh, wh)


NBUF = 2


def _make_agg_body(fw):
  def body(
      hp, rowh, colh, wh, out, rowv, colb, wb0, wb1, gb, acc, gsem, ssem, isem
  ):
    c = lax.axis_index("c")
    s = lax.axis_index("s")
    wid = s * NC + c
    base = wid * NCHW
    wbs = (wb0, wb1)

    pltpu.sync_copy(rowh.at[pl.ds(base, NCHW)], rowv)

    def zbody(i, _):
      for f in range(fw // L):
        gb[0, i, pl.ds(f * L, L)] = jnp.zeros((L,), jnp.float32)
      return 0

    lax.fori_loop(0, K, zbody, 0)
    for t in range(NZ):
      pltpu.sync_copy(gb.at[0], acc.at[pl.ds(s * RPS + t * K, K)])
    plsc.subcore_barrier()

    def g_start(b, cj):
      pltpu.async_copy(hp.at[rowv.at[cj]], gb.at[b], gsem.at[b])

    def g_wait(b, cj):
      pltpu.make_async_copy(hp.at[rowv.at[cj]], gb.at[b], gsem.at[b]).wait()

    def s_start(b, cj):
      pltpu.async_copy(gb.at[b], acc.at[colb.at[b]], ssem.at[b], add=True)

    def s_wait(b, cj):
      pltpu.make_async_copy(gb.at[b], acc.at[colb.at[b]], ssem.at[b]).wait()

    def i_start(b, cj):
      pltpu.async_copy(colh.at[base + cj], colb.at[b], isem.at[b])
      pltpu.async_copy(wh.at[base + cj], wbs[b], isem.at[b])

    def i_wait(b, cj):
      pltpu.make_async_copy(colh.at[base + cj], colb.at[b], isem.at[b]).wait()
      pltpu.make_async_copy(wh.at[base + cj], wbs[b], isem.at[b]).wait()

    i_start(0, 0)
    g_start(0, 0)

    def pair(t, _):
      j = t * NBUF
      for b in range(NBUF):
        cj = j + b
        bp = (b + NBUF - 1) % NBUF

        g_wait(b, cj)
        i_wait(b, cj)

        def edge(e, _):
          w16 = plsc.load_gather(wbs[b], [jnp.full((L,), e, jnp.int32)])
          for f in range(fw // L):
            sl = pl.ds(f * L, L)
            gb[b, e, sl] = gb[b, e, sl] * w16
          return 0

        lax.fori_loop(0, K, edge, 0, unroll=2)

        s_start(b, cj)

        @pl.when(cj >= 1)
        def _():
          s_wait(bp, cj - 1)

        @pl.when(cj + 1 < NCHW)
        def _():
          i_start(bp, cj + 1)
          g_start(bp, cj + 1)

      return 0

    lax.fori_loop(0, NCHW // NBUF, pair, 0)
    s_wait((NCHW - 1) % NBUF, NCHW - 1)
    plsc.subcore_barrier()

    sl = pl.ds(s * RPS, RPS)
    pltpu.sync_copy(acc.at[sl], out.at[c, sl])

  return body


def _make_agg(fw):
  @jax.jit
  def agg(hp, rowh, colh, wh):
    fn = pl.kernel(
        _make_agg_body(fw),
        out_type=jax.ShapeDtypeStruct((NC, NPAD, fw), jnp.float32),
        mesh=_mesh(),
        scratch_types=[
            pltpu.VMEM((NCHW, K), jnp.int32),
            pltpu.VMEM((NBUF, K), jnp.int32),
            pltpu.VMEM((K,), jnp.float32),
            pltpu.VMEM((K,), jnp.float32),
            pltpu.VMEM((NBUF, K, fw), jnp.float32),
            pltpu.VMEM_SHARED((NPAD, fw), jnp.float32),
            pltpu.SemaphoreType.DMA((NBUF,)),
            pltpu.SemaphoreType.DMA((NBUF,)),
            pltpu.SemaphoreType.DMA((NBUF,)),
        ],
        compiler_params=_sc_params(tc_tiling=(fw % 128 == 0)),
    )
    return fn(hp, rowh, colh, wh)

  return agg


_agg = _make_agg(NHID)
_agg64 = _make_agg(NCLASS)


def _dis(dp_ref):
  return lax.rsqrt(1.0 + dp_ref[:, 0:1] + dp_ref[:, 1:2])


def _tca_body(dp, x, w1, o):
  o[...] = _dis(dp) * jnp.dot(
      x[...], w1[...], preferred_element_type=jnp.float32
  )


def _tcb_body(dp, agg, hp, b1, w2, o):
  dis = _dis(dp)
  z = dis * (agg[0] + agg[1] + hp[...]) + b1[...][None, :]
  a = jnp.maximum(z, 0.0)
  o[...] = dis * jnp.dot(a, w2[...], preferred_element_type=jnp.float32)


def _tcc_body(dp, agg, hp, b2, o):
  dis = _dis(dp)
  o[...] = dis * (agg[0] + agg[1] + hp[...]) + b2[...][None, :]


def _agg_spec(fw):
  return pl.BlockSpec((NC, N, fw), lambda i: (0, 0, 0))


@jax.jit
def _tca(dp, x, w1):
  return pl.pallas_call(
      _tca_body,
      out_shape=jax.ShapeDtypeStruct((N, NHID), jnp.float32),
  )(dp, x, w1)


@jax.jit
def _tcb(dp, agg, hp, b1, w2):
  return pl.pallas_call(
      _tcb_body,
      grid=(1,),
      in_specs=[
          pl.BlockSpec((N, 2), lambda i: (0, 0)),
          _agg_spec(NHID),
          pl.BlockSpec((N, NHID), lambda i: (0, 0)),
          pl.BlockSpec((NHID,), lambda i: (0,)),
          pl.BlockSpec((NHID, NCLASS), lambda i: (0, 0)),
      ],
      out_specs=pl.BlockSpec((N, NCLASS), lambda i: (0, 0)),
      out_shape=jax.ShapeDtypeStruct((N, NCLASS), jnp.float32),
  )(dp, agg, hp, b1, w2)


@jax.jit
def _tcc(dp, agg, hp, b2):
  return pl.pallas_call(
      _tcc_body,
      grid=(1,),
      in_specs=[
          pl.BlockSpec((N, 2), lambda i: (0, 0)),
          _agg_spec(NCLASS),
          pl.BlockSpec((N, NCLASS), lambda i: (0, 0)),
          pl.BlockSpec((NCLASS,), lambda i: (0,)),
      ],
      out_specs=pl.BlockSpec((N, NCLASS), lambda i: (0, 0)),
      out_shape=jax.ShapeDtypeStruct((N, NCLASS), jnp.float32),
  )(dp, agg, hp, b2)


def kernel(x, edge_index, edge_weight, W1, b1, W2, b2):
  row = edge_index[0].astype(jnp.int32)
  col = edge_index[1].astype(jnp.int32)
  w = edge_weight.astype(jnp.float32)

  pad = EPAD - E
  pidx = (jnp.arange(pad, dtype=jnp.int32) * 997) % N
  row_p = jnp.concatenate([row, pidx]).reshape(NW * NCHW, K)
  col_p = jnp.concatenate([col, pidx]).reshape(NW * NCHW, K)
  w_p = jnp.concatenate([w, jnp.zeros((pad,), jnp.float32)]).reshape(
      NW * NCHW, K
  )

  d0, d1 = _deg(col_p, w_p)
  dp = jnp.stack([d0[:N], d1[:N]], axis=1)

  h1p = _tca(dp, x, W1)
  agg1 = _agg(h1p, row_p, col_p, w_p)
  h2p = _tcb(dp, agg1, h1p, b1, W2)
  agg2 = _agg64(h2p, row_p, col_p, w_p)
  out = _tcc(dp, agg2, h2p, b2)
  return out
